# baseline (speedup 1.0000x reference)
import functools

import jax
import jax.numpy as jnp
from jax import lax
from jax.experimental import pallas as pl
from jax.experimental.pallas import tpu as pltpu
from jax.experimental.pallas import tpu_sc as plsc

F32 = jnp.float32

C = 256
CTX_UP = 2
UP = 4
GRID = 4
TGT_CTX = 1024
TGT_TGT = 6048


def _fixed_noise():
    import numpy as np
    cpu = jax.devices("cpu")[0]
    with jax.default_device(cpu):
        key = jax.random.key(42)
        n_ctx = 0.0001 * jax.random.normal(jax.random.fold_in(key, 0),
                                           (1, 1024, 3), dtype=F32)
        x1 = 0.02 * jax.random.normal(jax.random.fold_in(key, 100),
                                      (1, 2048, 3), F32)
        x2 = 0.02 * jax.random.normal(jax.random.fold_in(key, 101),
                                      (1, 8192, 3), F32)
        t1 = 0.01 * jax.random.normal(jax.random.fold_in(key, 200),
                                      (1, 2048, C), F32)
        t2 = 0.01 * jax.random.normal(jax.random.fold_in(key, 201),
                                      (1, 8192, C), F32)
        return tuple(np.asarray(a) for a in (n_ctx, x1, x2, t1, t2))


_N_CTX, _N_X1, _N_X2, _N_T1, _N_T2 = _fixed_noise()


_BIG = 1e30


def _tree_min(v, payloads):
    while v.shape[1] > 1:
        h = v.shape[1] // 2
        ta = v[:, :h] <= v[:, h:]
        v = jnp.where(ta, v[:, :h], v[:, h:])
        payloads = [jnp.where(ta, p[:, :h], p[:, h:]) for p in payloads]
    return v, payloads


def _knn_body(k, RB, N, P, W, WP, xb_ref, xft_ref, idx_ref):
    i = pl.program_id(0)
    Np = P * W
    xb = xb_ref[...]
    xft = xft_ref[...]
    d0 = xb[:, 0:1] - xft[0:1, :]
    d1 = xb[:, 1:2] - xft[1:2, :]
    d2 = xb[:, 2:3] - xft[2:3, :]
    D = d0 * d0 + d1 * d1 + d2 * d2
    rows = i * RB + lax.broadcasted_iota(jnp.int32, (RB, Np), 0)
    cols = lax.broadcasted_iota(jnp.int32, (RB, Np), 1)
    bad = (rows == cols) | (cols >= N) if Np > N else (rows == cols)
    D = jnp.where(bad, _BIG, D)

    planes = [D[:, m * W:(m + 1) * W] for m in range(P)]
    M = planes[0]
    for m in range(1, P):
        M = jnp.minimum(M, planes[m])
    Q = lax.broadcasted_iota(jnp.int32, (RB, W), 1)
    if WP > W:
        padf = jnp.full((RB, WP - W), _BIG, F32)
        padi = jnp.zeros((RB, WP - W), jnp.int32)
        M = jnp.concatenate([M, padf], axis=1)
        Q = jnp.concatenate([Q, padi], axis=1)
        planes = [jnp.concatenate([p, padf], axis=1) for p in planes]

    q_outs, v_outs = [], []
    for _ in range(k):
        _, pay = _tree_min(M, [Q] + planes)
        qt = pay[0]
        q_outs.append(qt)
        v_outs.append(pay[1:])
        M = jnp.where(Q == qt, _BIG, M)

    cv = jnp.concatenate([v for vs in v_outs for v in vs], axis=1)
    cc = jnp.concatenate(
        [qt + m * W for qt in q_outs for m in range(P)], axis=1)
    outs = []
    for _ in range(k):
        _, (ct,) = _tree_min(cv, [cc])
        outs.append(ct)
        cv = jnp.where(cc == ct, _BIG, cv)
    idx_ref[...] = jnp.concatenate(outs, axis=1)


def _knn(xyz, k):
    N = xyz.shape[0]
    P = 8
    W = -(-N // (P * 128)) * 128
    WP = 1 << (W - 1).bit_length()
    Np = P * W
    xp = jnp.zeros((N, 8), F32).at[:, :3].set(xyz)
    xpt = jnp.zeros((8, Np), F32).at[:3, :N].set(xyz.T)
    RB = 224 if N % 224 == 0 else 128
    grid = N // RB
    return pl.pallas_call(
        functools.partial(_knn_body, k, RB, N, P, W, WP),
        grid=(grid,),
        in_specs=[
            pl.BlockSpec((RB, 8), lambda i: (i, 0)),
            pl.BlockSpec((8, Np), lambda i: (0, 0)),
        ],
        out_specs=pl.BlockSpec((RB, k), lambda i: (i, 0)),
        out_shape=jax.ShapeDtypeStruct((N, k), jnp.int32),
    )(xp, xpt)


_SC_CHUNK = 128


def _gather_rows(table, idx):
    V, H = table.shape
    E = idx.shape[0]
    NW = 32
    b_per_w = E // NW
    ch = 64 if H >= 256 else _SC_CHUNK
    n_ch = b_per_w // ch
    nbuf = min(4, n_ch)
    look = min(2, n_ch - 1)
    mesh = plsc.VectorSubcoreMesh(core_axis_name="c", subcore_axis_name="s")

    @functools.partial(
        pl.kernel,
        mesh=mesh,
        out_type=jax.ShapeDtypeStruct((E, H), F32),
        scratch_types=[
            pltpu.VMEM((b_per_w,), jnp.int32),
            pltpu.VMEM((nbuf, ch, H), F32),
        ]
        + [pltpu.SemaphoreType.DMA] * (2 * nbuf),
    )
    def gk(table_hbm, idx_hbm, out_hbm, idx_v, ring_v, *sems):
        gsem, ssem = sems[:nbuf], sems[nbuf:]
        wid = lax.axis_index("s") * 2 + lax.axis_index("c")
        base = wid * b_per_w
        pltpu.sync_copy(idx_hbm.at[pl.ds(base, b_per_w)], idx_v)
        gathers = [None] * n_ch
        stores = [None] * n_ch

        def fire_store(pj):
            stores[pj] = pltpu.async_copy(
                ring_v.at[pj % nbuf],
                out_hbm.at[pl.ds(base + pj * ch, ch)],
                ssem[pj % nbuf])

        for j in range(n_ch):
            b = j % nbuf
            if j >= nbuf:
                stores[j - nbuf].wait()
            gathers[j] = pltpu.async_copy(
                table_hbm.at[idx_v.at[pl.ds(j * ch, ch)]],
                ring_v.at[b], gsem[b])
            if j >= look:
                gathers[j - look].wait()
                fire_store(j - look)
        for j in range(n_ch - look, n_ch):
            gathers[j].wait()
            fire_store(j)
        for j in range(max(0, n_ch - nbuf), n_ch):
            stores[j].wait()

    return gk(table, idx)


def _gather_neighbors(table, idx2d):
    N, K = idx2d.shape
    H = table.shape[1]
    flat = idx2d.reshape(-1)
    E = flat.shape[0]
    Ep = ((E + 4095) // 4096) * 4096
    if Ep != E:
        flat = jnp.concatenate([flat, jnp.zeros((Ep - E,), jnp.int32)])
    rows = _gather_rows(table, flat)
    return rows[:E].reshape(N, K, H)


def _edge_body(NB, K, H, Ho, off, has_res, *refs):
    if has_res:
        bj_ref, cn_ref, w2_ref, b2_ref, res_ref, out_ref = refs
    else:
        bj_ref, cn_ref, w2_ref, b2_ref, out_ref = refs
    bj = bj_ref[...][:, :, off:off + H]
    e = jnp.maximum(cn_ref[...][:, None, :] + bj, 0.0)
    h = lax.dot_general(e.reshape(NB * K, H), w2_ref[...],
                        (((1,), (0,)), ((), ())), preferred_element_type=F32)
    r = jnp.max(h.reshape(NB, K, Ho), axis=1) + b2_ref[0:1, :]
    if has_res:
        r = r + res_ref[...]
    out_ref[...] = r


def _edge(bj, cn, W2, b2, off=0, resid=None):
    N, K, Hb = bj.shape
    H = cn.shape[1]
    Ho = W2.shape[1]
    NB = 504 if N % 504 == 0 else 256
    grid = N // NB
    b2b = jnp.broadcast_to(b2[None, :], (8, Ho))
    in_specs = [
        pl.BlockSpec((NB, K, Hb), lambda i: (i, 0, 0)),
        pl.BlockSpec((NB, H), lambda i: (i, 0)),
        pl.BlockSpec((H, Ho), lambda i: (0, 0)),
        pl.BlockSpec((8, Ho), lambda i: (0, 0)),
    ]
    args = [bj, cn, W2, b2b]
    if resid is not None:
        in_specs.append(pl.BlockSpec((NB, Ho), lambda i: (i, 0)))
        args.append(resid)
    return pl.pallas_call(
        functools.partial(_edge_body, NB, K, H, Ho, off, resid is not None),
        grid=(grid,),
        in_specs=in_specs,
        out_specs=pl.BlockSpec((NB, Ho), lambda i: (i, 0)),
        out_shape=jax.ShapeDtypeStruct((N, Ho), F32),
    )(*args)


def _mm_body(relu, x_ref, w_ref, b_ref, out_ref):
    h = lax.dot_general(x_ref[...], w_ref[...], (((1,), (0,)), ((), ())),
                        preferred_element_type=F32) + b_ref[0:1, :]
    if relu:
        h = jnp.maximum(h, 0.0)
    out_ref[...] = h


def _mm(X, W, b, relu=False):
    N, F = X.shape
    Ho = W.shape[1]
    NB = 1008 if N % 1008 == 0 else (512 if N % 512 == 0 else 256)
    grid = N // NB
    b2b = jnp.broadcast_to(b[None, :], (8, Ho))
    return pl.pallas_call(
        functools.partial(_mm_body, relu),
        grid=(grid,),
        in_specs=[
            pl.BlockSpec((NB, F), lambda i: (i, 0)),
            pl.BlockSpec((F, Ho), lambda i: (0, 0)),
            pl.BlockSpec((8, Ho), lambda i: (0, 0)),
        ],
        out_specs=pl.BlockSpec((NB, Ho), lambda i: (i, 0)),
        out_shape=jax.ShapeDtypeStruct((N, Ho), F32),
    )(X, W, b2b)


def _mlp2_body(has_res, x_ref, w1_ref, b1_ref, w2_ref, b2_ref, *rest):
    if has_res:
        res_ref, out_ref = rest
    else:
        (out_ref,) = rest
    h = lax.dot_general(x_ref[...], w1_ref[...], (((1,), (0,)), ((), ())),
                        preferred_element_type=F32) + b1_ref[0:1, :]
    h = jnp.maximum(h, 0.0)
    o = lax.dot_general(h, w2_ref[...], (((1,), (0,)), ((), ())),
                        preferred_element_type=F32) + b2_ref[0:1, :]
    if has_res:
        o = o + res_ref[...]
    out_ref[...] = o


def _mlp2(X, W1, b1, W2, b2, resid=None):
    N, F = X.shape
    Hh = W1.shape[1]
    Ho = W2.shape[1]
    NB = 1008 if N % 1008 == 0 else (512 if N % 512 == 0 else 256)
    grid = N // NB
    b1b = jnp.broadcast_to(b1[None, :], (8, Hh))
    b2b = jnp.broadcast_to(b2[None, :], (8, Ho))
    in_specs = [
        pl.BlockSpec((NB, F), lambda i: (i, 0)),
        pl.BlockSpec((F, Hh), lambda i: (0, 0)),
        pl.BlockSpec((8, Hh), lambda i: (0, 0)),
        pl.BlockSpec((Hh, Ho), lambda i: (0, 0)),
        pl.BlockSpec((8, Ho), lambda i: (0, 0)),
    ]
    args = [X, W1, b1b, W2, b2b]
    if resid is not None:
        in_specs.append(pl.BlockSpec((NB, Ho), lambda i: (i, 0)))
        args.append(resid)
    return pl.pallas_call(
        functools.partial(_mlp2_body, resid is not None),
        grid=(grid,),
        in_specs=in_specs,
        out_specs=pl.BlockSpec((NB, Ho), lambda i: (i, 0)),
        out_shape=jax.ShapeDtypeStruct((N, Ho), F32),
    )(*args)


def _edgeconv(x, idx, W1, b1, W2, b2, resid=None):
    F = x.shape[1]
    H = W1.shape[1]
    W1a, W1b = W1[:F], W1[F:]
    Wcat = jnp.concatenate([W1a - W1b, W1b], axis=1)
    bcat = jnp.concatenate([b1, jnp.zeros((H,), F32)])
    cb = _mm(x, Wcat, bcat)
    cn = cb[:, :H]
    if H % 128 == 0:
        bj = _gather_neighbors(cb[:, H:], idx)
        off = 0
    else:
        bj = _gather_neighbors(cb, idx)
        off = H
    return _edge(bj, cn, W2, b2, off=off, resid=resid)


def kernel(ctx_xyz, ctx_tokens, pred_tokens, params, mask_id):
    p = params

    cxyz = (jnp.repeat(ctx_xyz, CTX_UP, axis=1) +
            jnp.asarray(_N_CTX)).reshape(-1, 3)
    ctok = jnp.repeat(ctx_tokens, CTX_UP, axis=1).reshape(-1, C)

    idx_c = _knn(cxyz, 16)
    x0 = jnp.concatenate([ctok, cxyz], axis=-1)
    x1 = _edgeconv(x0, idx_c, p['ctx1_W1'], p['ctx1_b1'],
                   p['ctx1_W2'], p['ctx1_b2'])
    x1c = jnp.concatenate([x1, cxyz], axis=-1)
    ctx_feat = _edgeconv(x1c, idx_c, p['ctx2_W1'], p['ctx2_b1'],
                         p['ctx2_W2'], p['ctx2_b2'])
    ctx_out = _mlp2(jnp.concatenate([cxyz, ctx_feat], axis=-1),
                    p['def_W1'], p['def_b1'],
                    0.05 * p['def_W2'], 0.05 * p['def_b2'], resid=cxyz)

    pred_tok = pred_tokens[:, mask_id]
    tgt_xyz = _mlp2(pred_tok.reshape(-1, C), p['lat_W1'], p['lat_b1'],
                    p['lat_W2'], p['lat_b2'])[None]
    txyz = (jnp.repeat(jnp.repeat(tgt_xyz, UP, axis=1) + jnp.asarray(_N_X1),
                       UP, axis=1) + jnp.asarray(_N_X2))[:, :TGT_TGT]
    txyz = txyz.reshape(-1, 3)
    ttok = (jnp.repeat(jnp.repeat(pred_tok, UP, axis=1) + jnp.asarray(_N_T1),
                       UP, axis=1) + jnp.asarray(_N_T2))[:, :TGT_TGT]
    ttok = ttok.reshape(-1, C)

    idx16 = _knn(txyz, 16)
    x = _edgeconv(ttok, idx16, p['te1_W1'], p['te1_b1'],
                  p['te1_W2'], p['te1_b2'])
    tgt_feat = _edgeconv(x, idx16, p['te2_W1'], p['te2_b1'],
                         p['te2_W2'], p['te2_b2'])

    g = jnp.linspace(-1.0, 1.0, GRID)
    grid = jnp.stack(jnp.meshgrid(g, g, indexing='ij'), axis=-1).reshape(-1, 2)
    N = tgt_feat.shape[0]
    grid = jnp.tile(grid, (N // grid.shape[0] + 1, 1))[:N].astype(F32)
    xyz = _mlp2(jnp.concatenate([grid, tgt_feat], axis=-1),
                p['f1_W1'], p['f1_b1'], p['f1_W2'], p['f1_b2'])
    xyz = _mlp2(jnp.concatenate([xyz, tgt_feat], axis=-1),
                p['f2_W1'], p['f2_b1'], p['f2_W2'], p['f2_b2'], resid=xyz)

    idx8 = _knn(xyz, 8)
    xf = jnp.concatenate([tgt_feat, xyz], axis=-1)
    xyz = _edgeconv(xf, idx8, p['ref_W1'], p['ref_b1'],
                    p['ref_W2'], p['ref_b2'], resid=xyz)
    return jnp.concatenate([ctx_out, xyz], axis=0)


# speedup vs baseline: 1.9778x; 1.9778x over previous
import functools

import jax
import jax.numpy as jnp
from jax import lax
from jax.experimental import pallas as pl
from jax.experimental.pallas import tpu as pltpu
from jax.experimental.pallas import tpu_sc as plsc

F32 = jnp.float32

C = 256
CTX_UP = 2
UP = 4
GRID = 4
TGT_CTX = 1024
TGT_TGT = 6048


def _fixed_noise():
    import numpy as np
    cpu = jax.devices("cpu")[0]
    with jax.default_device(cpu):
        key = jax.random.key(42)
        n_ctx = 0.0001 * jax.random.normal(jax.random.fold_in(key, 0),
                                           (1, 1024, 3), dtype=F32)
        x1 = 0.02 * jax.random.normal(jax.random.fold_in(key, 100),
                                      (1, 2048, 3), F32)
        x2 = 0.02 * jax.random.normal(jax.random.fold_in(key, 101),
                                      (1, 8192, 3), F32)
        t1 = 0.01 * jax.random.normal(jax.random.fold_in(key, 200),
                                      (1, 2048, C), F32)
        t2 = 0.01 * jax.random.normal(jax.random.fold_in(key, 201),
                                      (1, 8192, C), F32)
        return tuple(np.asarray(a) for a in (n_ctx, x1, x2, t1, t2))


_N_CTX, _N_X1, _N_X2, _N_T1, _N_T2 = _fixed_noise()


_BIG = 1e30


def _tree_min(v, payloads):
    while v.shape[1] > 1:
        h = v.shape[1] // 2
        ta = v[:, :h] <= v[:, h:]
        v = jnp.where(ta, v[:, :h], v[:, h:])
        payloads = [jnp.where(ta, p[:, :h], p[:, h:]) for p in payloads]
    return v, payloads


def _knn_body(k, RB, N, P, W, WP, xb_ref, xft_ref, idx_ref):
    i = pl.program_id(0)
    Np = P * W
    xb = xb_ref[...]
    xft = xft_ref[...]
    d0 = xb[:, 0:1] - xft[0:1, :]
    d1 = xb[:, 1:2] - xft[1:2, :]
    d2 = xb[:, 2:3] - xft[2:3, :]
    D = d0 * d0 + d1 * d1 + d2 * d2
    rows = i * RB + lax.broadcasted_iota(jnp.int32, (RB, Np), 0)
    cols = lax.broadcasted_iota(jnp.int32, (RB, Np), 1)
    bad = (rows == cols) | (cols >= N) if Np > N else (rows == cols)
    D = jnp.where(bad, _BIG, D)

    planes = [D[:, m * W:(m + 1) * W] for m in range(P)]
    M = planes[0]
    for m in range(1, P):
        M = jnp.minimum(M, planes[m])
    Q = lax.broadcasted_iota(jnp.int32, (RB, W), 1)
    if WP > W:
        padf = jnp.full((RB, WP - W), _BIG, F32)
        padi = jnp.zeros((RB, WP - W), jnp.int32)
        M = jnp.concatenate([M, padf], axis=1)
        Q = jnp.concatenate([Q, padi], axis=1)
        planes = [jnp.concatenate([p, padf], axis=1) for p in planes]

    q_outs, v_outs = [], []
    for _ in range(k):
        qt = jnp.argmin(M, axis=1).astype(jnp.int32)[:, None]
        oh = Q == qt
        q_outs.append(qt)
        v_outs.append([jnp.min(jnp.where(oh, p, _BIG), axis=1)[:, None]
                       for p in planes])
        M = jnp.where(oh, _BIG, M)

    cv = jnp.concatenate([v for vs in v_outs for v in vs], axis=1)
    cc = jnp.concatenate(
        [qt + m * W for qt in q_outs for m in range(P)], axis=1)
    pos = lax.broadcasted_iota(jnp.int32, (RB, k * P), 1)
    outs = []
    for _ in range(k):
        am2 = jnp.argmin(cv, axis=1).astype(jnp.int32)[:, None]
        oh2 = pos == am2
        outs.append(jnp.max(jnp.where(oh2, cc, -1), axis=1)[:, None])
        cv = jnp.where(oh2, _BIG, cv)
    idx_ref[...] = jnp.concatenate(outs, axis=1)


def _knn(xyz, k):
    N = xyz.shape[0]
    P = 8
    W = -(-N // (P * 128)) * 128
    WP = 1 << (W - 1).bit_length()
    Np = P * W
    xp = jnp.zeros((N, 8), F32).at[:, :3].set(xyz)
    xpt = jnp.zeros((8, Np), F32).at[:3, :N].set(xyz.T)
    RB = 224 if N % 224 == 0 else 128
    grid = N // RB
    return pl.pallas_call(
        functools.partial(_knn_body, k, RB, N, P, W, WP),
        grid=(grid,),
        in_specs=[
            pl.BlockSpec((RB, 8), lambda i: (i, 0)),
            pl.BlockSpec((8, Np), lambda i: (0, 0)),
        ],
        out_specs=pl.BlockSpec((RB, k), lambda i: (i, 0)),
        out_shape=jax.ShapeDtypeStruct((N, k), jnp.int32),
    )(xp, xpt)


_SC_CHUNK = 128


def _gather_rows(table, idx):
    V, H = table.shape
    E = idx.shape[0]
    NW = 32
    b_per_w = E // NW
    ch = 64 if H >= 256 else _SC_CHUNK
    n_ch = b_per_w // ch
    nbuf = min(4, n_ch)
    look = min(2, n_ch - 1)
    mesh = plsc.VectorSubcoreMesh(core_axis_name="c", subcore_axis_name="s")

    @functools.partial(
        pl.kernel,
        mesh=mesh,
        out_type=jax.ShapeDtypeStruct((E, H), F32),
        scratch_types=[
            pltpu.VMEM((b_per_w,), jnp.int32),
            pltpu.VMEM((nbuf, ch, H), F32),
        ]
        + [pltpu.SemaphoreType.DMA] * (2 * nbuf),
    )
    def gk(table_hbm, idx_hbm, out_hbm, idx_v, ring_v, *sems):
        gsem, ssem = sems[:nbuf], sems[nbuf:]
        wid = lax.axis_index("s") * 2 + lax.axis_index("c")
        base = wid * b_per_w
        pltpu.sync_copy(idx_hbm.at[pl.ds(base, b_per_w)], idx_v)
        gathers = [None] * n_ch
        stores = [None] * n_ch

        def fire_store(pj):
            stores[pj] = pltpu.async_copy(
                ring_v.at[pj % nbuf],
                out_hbm.at[pl.ds(base + pj * ch, ch)],
                ssem[pj % nbuf])

        for j in range(n_ch):
            b = j % nbuf
            if j >= nbuf:
                stores[j - nbuf].wait()
            gathers[j] = pltpu.async_copy(
                table_hbm.at[idx_v.at[pl.ds(j * ch, ch)]],
                ring_v.at[b], gsem[b])
            if j >= look:
                gathers[j - look].wait()
                fire_store(j - look)
        for j in range(n_ch - look, n_ch):
            gathers[j].wait()
            fire_store(j)
        for j in range(max(0, n_ch - nbuf), n_ch):
            stores[j].wait()

    return gk(table, idx)


def _gather_neighbors(table, idx2d):
    N, K = idx2d.shape
    H = table.shape[1]
    flat = idx2d.reshape(-1)
    E = flat.shape[0]
    Ep = ((E + 4095) // 4096) * 4096
    if Ep != E:
        flat = jnp.concatenate([flat, jnp.zeros((Ep - E,), jnp.int32)])
    rows = _gather_rows(table, flat)
    return rows[:E].reshape(N, K, H)


def _edge_body(NB, K, H, Ho, off, has_res, *refs):
    if has_res:
        bj_ref, cn_ref, w2_ref, b2_ref, res_ref, out_ref = refs
    else:
        bj_ref, cn_ref, w2_ref, b2_ref, out_ref = refs
    bj = bj_ref[...][:, :, off:off + H]
    e = jnp.maximum(cn_ref[...][:, None, :] + bj, 0.0)
    h = lax.dot_general(e.reshape(NB * K, H), w2_ref[...],
                        (((1,), (0,)), ((), ())), preferred_element_type=F32)
    r = jnp.max(h.reshape(NB, K, Ho), axis=1) + b2_ref[0:1, :]
    if has_res:
        r = r + res_ref[...]
    out_ref[...] = r


def _edge(bj, cn, W2, b2, off=0, resid=None):
    N, K, Hb = bj.shape
    H = cn.shape[1]
    Ho = W2.shape[1]
    NB = 504 if N % 504 == 0 else 256
    grid = N // NB
    b2b = jnp.broadcast_to(b2[None, :], (8, Ho))
    in_specs = [
        pl.BlockSpec((NB, K, Hb), lambda i: (i, 0, 0)),
        pl.BlockSpec((NB, H), lambda i: (i, 0)),
        pl.BlockSpec((H, Ho), lambda i: (0, 0)),
        pl.BlockSpec((8, Ho), lambda i: (0, 0)),
    ]
    args = [bj, cn, W2, b2b]
    if resid is not None:
        in_specs.append(pl.BlockSpec((NB, Ho), lambda i: (i, 0)))
        args.append(resid)
    return pl.pallas_call(
        functools.partial(_edge_body, NB, K, H, Ho, off, resid is not None),
        grid=(grid,),
        in_specs=in_specs,
        out_specs=pl.BlockSpec((NB, Ho), lambda i: (i, 0)),
        out_shape=jax.ShapeDtypeStruct((N, Ho), F32),
    )(*args)


def _mm_body(relu, x_ref, w_ref, b_ref, out_ref):
    h = lax.dot_general(x_ref[...], w_ref[...], (((1,), (0,)), ((), ())),
                        preferred_element_type=F32) + b_ref[0:1, :]
    if relu:
        h = jnp.maximum(h, 0.0)
    out_ref[...] = h


def _mm(X, W, b, relu=False):
    N, F = X.shape
    Ho = W.shape[1]
    NB = 1008 if N % 1008 == 0 else (512 if N % 512 == 0 else 256)
    grid = N // NB
    b2b = jnp.broadcast_to(b[None, :], (8, Ho))
    return pl.pallas_call(
        functools.partial(_mm_body, relu),
        grid=(grid,),
        in_specs=[
            pl.BlockSpec((NB, F), lambda i: (i, 0)),
            pl.BlockSpec((F, Ho), lambda i: (0, 0)),
            pl.BlockSpec((8, Ho), lambda i: (0, 0)),
        ],
        out_specs=pl.BlockSpec((NB, Ho), lambda i: (i, 0)),
        out_shape=jax.ShapeDtypeStruct((N, Ho), F32),
    )(X, W, b2b)


def _mlp2_body(has_res, x_ref, w1_ref, b1_ref, w2_ref, b2_ref, *rest):
    if has_res:
        res_ref, out_ref = rest
    else:
        (out_ref,) = rest
    h = lax.dot_general(x_ref[...], w1_ref[...], (((1,), (0,)), ((), ())),
                        preferred_element_type=F32) + b1_ref[0:1, :]
    h = jnp.maximum(h, 0.0)
    o = lax.dot_general(h, w2_ref[...], (((1,), (0,)), ((), ())),
                        preferred_element_type=F32) + b2_ref[0:1, :]
    if has_res:
        o = o + res_ref[...]
    out_ref[...] = o


def _mlp2(X, W1, b1, W2, b2, resid=None):
    N, F = X.shape
    Hh = W1.shape[1]
    Ho = W2.shape[1]
    NB = 1008 if N % 1008 == 0 else (512 if N % 512 == 0 else 256)
    grid = N // NB
    b1b = jnp.broadcast_to(b1[None, :], (8, Hh))
    b2b = jnp.broadcast_to(b2[None, :], (8, Ho))
    in_specs = [
        pl.BlockSpec((NB, F), lambda i: (i, 0)),
        pl.BlockSpec((F, Hh), lambda i: (0, 0)),
        pl.BlockSpec((8, Hh), lambda i: (0, 0)),
        pl.BlockSpec((Hh, Ho), lambda i: (0, 0)),
        pl.BlockSpec((8, Ho), lambda i: (0, 0)),
    ]
    args = [X, W1, b1b, W2, b2b]
    if resid is not None:
        in_specs.append(pl.BlockSpec((NB, Ho), lambda i: (i, 0)))
        args.append(resid)
    return pl.pallas_call(
        functools.partial(_mlp2_body, resid is not None),
        grid=(grid,),
        in_specs=in_specs,
        out_specs=pl.BlockSpec((NB, Ho), lambda i: (i, 0)),
        out_shape=jax.ShapeDtypeStruct((N, Ho), F32),
    )(*args)


def _edgeconv(x, idx, W1, b1, W2, b2, resid=None):
    F = x.shape[1]
    H = W1.shape[1]
    W1a, W1b = W1[:F], W1[F:]
    Wcat = jnp.concatenate([W1a - W1b, W1b], axis=1)
    bcat = jnp.concatenate([b1, jnp.zeros((H,), F32)])
    cb = _mm(x, Wcat, bcat)
    cn = cb[:, :H]
    if H % 128 == 0:
        bj = _gather_neighbors(cb[:, H:], idx)
        off = 0
    else:
        bj = _gather_neighbors(cb, idx)
        off = H
    return _edge(bj, cn, W2, b2, off=off, resid=resid)


def kernel(ctx_xyz, ctx_tokens, pred_tokens, params, mask_id):
    p = params

    cxyz = (jnp.repeat(ctx_xyz, CTX_UP, axis=1) +
            jnp.asarray(_N_CTX)).reshape(-1, 3)
    ctok = jnp.repeat(ctx_tokens, CTX_UP, axis=1).reshape(-1, C)

    idx_c = _knn(cxyz, 16)
    x0 = jnp.concatenate([ctok, cxyz], axis=-1)
    x1 = _edgeconv(x0, idx_c, p['ctx1_W1'], p['ctx1_b1'],
                   p['ctx1_W2'], p['ctx1_b2'])
    x1c = jnp.concatenate([x1, cxyz], axis=-1)
    ctx_feat = _edgeconv(x1c, idx_c, p['ctx2_W1'], p['ctx2_b1'],
                         p['ctx2_W2'], p['ctx2_b2'])
    ctx_out = _mlp2(jnp.concatenate([cxyz, ctx_feat], axis=-1),
                    p['def_W1'], p['def_b1'],
                    0.05 * p['def_W2'], 0.05 * p['def_b2'], resid=cxyz)

    pred_tok = pred_tokens[:, mask_id]
    tgt_xyz = _mlp2(pred_tok.reshape(-1, C), p['lat_W1'], p['lat_b1'],
                    p['lat_W2'], p['lat_b2'])[None]
    txyz = (jnp.repeat(jnp.repeat(tgt_xyz, UP, axis=1) + jnp.asarray(_N_X1),
                       UP, axis=1) + jnp.asarray(_N_X2))[:, :TGT_TGT]
    txyz = txyz.reshape(-1, 3)
    ttok = (jnp.repeat(jnp.repeat(pred_tok, UP, axis=1) + jnp.asarray(_N_T1),
                       UP, axis=1) + jnp.asarray(_N_T2))[:, :TGT_TGT]
    ttok = ttok.reshape(-1, C)

    idx16 = _knn(txyz, 16)
    x = _edgeconv(ttok, idx16, p['te1_W1'], p['te1_b1'],
                  p['te1_W2'], p['te1_b2'])
    tgt_feat = _edgeconv(x, idx16, p['te2_W1'], p['te2_b1'],
                         p['te2_W2'], p['te2_b2'])

    g = jnp.linspace(-1.0, 1.0, GRID)
    grid = jnp.stack(jnp.meshgrid(g, g, indexing='ij'), axis=-1).reshape(-1, 2)
    N = tgt_feat.shape[0]
    grid = jnp.tile(grid, (N // grid.shape[0] + 1, 1))[:N].astype(F32)
    xyz = _mlp2(jnp.concatenate([grid, tgt_feat], axis=-1),
                p['f1_W1'], p['f1_b1'], p['f1_W2'], p['f1_b2'])
    xyz = _mlp2(jnp.concatenate([xyz, tgt_feat], axis=-1),
                p['f2_W1'], p['f2_b1'], p['f2_W2'], p['f2_b2'], resid=xyz)

    idx8 = _knn(xyz, 8)
    xf = jnp.concatenate([tgt_feat, xyz], axis=-1)
    xyz = _edgeconv(xf, idx8, p['ref_W1'], p['ref_b1'],
                    p['ref_W2'], p['ref_b2'], resid=xyz)
    return jnp.concatenate([ctx_out, xyz], axis=0)


# speedup vs baseline: 2.3172x; 1.1716x over previous
import functools

import jax
import jax.numpy as jnp
from jax import lax
from jax.experimental import pallas as pl
from jax.experimental.pallas import tpu as pltpu
from jax.experimental.pallas import tpu_sc as plsc

F32 = jnp.float32

C = 256
CTX_UP = 2
UP = 4
GRID = 4
TGT_CTX = 1024
TGT_TGT = 6048


def _fixed_noise():
    import numpy as np
    cpu = jax.devices("cpu")[0]
    with jax.default_device(cpu):
        key = jax.random.key(42)
        n_ctx = 0.0001 * jax.random.normal(jax.random.fold_in(key, 0),
                                           (1, 1024, 3), dtype=F32)
        x1 = 0.02 * jax.random.normal(jax.random.fold_in(key, 100),
                                      (1, 2048, 3), F32)
        x2 = 0.02 * jax.random.normal(jax.random.fold_in(key, 101),
                                      (1, 8192, 3), F32)
        t1 = 0.01 * jax.random.normal(jax.random.fold_in(key, 200),
                                      (1, 2048, C), F32)
        t2 = 0.01 * jax.random.normal(jax.random.fold_in(key, 201),
                                      (1, 8192, C), F32)
        return tuple(np.asarray(a) for a in (n_ctx, x1, x2, t1, t2))


_N_CTX, _N_X1, _N_X2, _N_T1, _N_T2 = _fixed_noise()


def _knn_body(k, RB, N, xb_ref, xft_ref, idx_ref):
    i = pl.program_id(0)
    xb = xb_ref[...]
    xft = xft_ref[...]
    d0 = xb[:, 0:1] - xft[0:1, :]
    d1 = xb[:, 1:2] - xft[1:2, :]
    d2 = xb[:, 2:3] - xft[2:3, :]
    D = d0 * d0 + d1 * d1 + d2 * d2
    rows = i * RB + lax.broadcasted_iota(jnp.int32, (RB, N), 0)
    cols = lax.broadcasted_iota(jnp.int32, (RB, N), 1)
    D = jnp.where(rows == cols, 1e30, D)
    outs = []
    for _ in range(k):
        am = jnp.argmin(D, axis=1).astype(jnp.int32)
        outs.append(am)
        D = jnp.where(cols == am[:, None], 1e30, D)
    idx_ref[...] = jnp.stack(outs, axis=1)


def _knn(xyz, k):
    N = xyz.shape[0]
    xp = jnp.zeros((N, 8), F32).at[:, :3].set(xyz)
    xpt = jnp.zeros((8, N), F32).at[:3, :].set(xyz.T)
    RB = 224 if N % 224 == 0 else 128
    grid = N // RB
    return pl.pallas_call(
        functools.partial(_knn_body, k, RB, N),
        grid=(grid,),
        in_specs=[
            pl.BlockSpec((RB, 8), lambda i: (i, 0)),
            pl.BlockSpec((8, N), lambda i: (0, 0)),
        ],
        out_specs=pl.BlockSpec((RB, k), lambda i: (i, 0)),
        out_shape=jax.ShapeDtypeStruct((N, k), jnp.int32),
    )(xp, xpt)


_SC_CHUNK = 128


def _gather_rows(table, idx):
    V, H = table.shape
    E = idx.shape[0]
    NW = 32
    b_per_w = E // NW
    ch = 64 if H >= 256 else _SC_CHUNK
    n_ch = b_per_w // ch
    nbuf = min(4, n_ch)
    look = min(2, n_ch - 1)
    mesh = plsc.VectorSubcoreMesh(core_axis_name="c", subcore_axis_name="s")

    @functools.partial(
        pl.kernel,
        mesh=mesh,
        out_type=jax.ShapeDtypeStruct((E, H), F32),
        scratch_types=[
            pltpu.VMEM((b_per_w,), jnp.int32),
            pltpu.VMEM((nbuf, ch, H), F32),
        ]
        + [pltpu.SemaphoreType.DMA] * (2 * nbuf),
    )
    def gk(table_hbm, idx_hbm, out_hbm, idx_v, ring_v, *sems):
        gsem, ssem = sems[:nbuf], sems[nbuf:]
        wid = lax.axis_index("s") * 2 + lax.axis_index("c")
        base = wid * b_per_w
        pltpu.sync_copy(idx_hbm.at[pl.ds(base, b_per_w)], idx_v)
        gathers = [None] * n_ch
        stores = [None] * n_ch

        def fire_store(pj):
            stores[pj] = pltpu.async_copy(
                ring_v.at[pj % nbuf],
                out_hbm.at[pl.ds(base + pj * ch, ch)],
                ssem[pj % nbuf])

        for j in range(n_ch):
            b = j % nbuf
            if j >= nbuf:
                stores[j - nbuf].wait()
            gathers[j] = pltpu.async_copy(
                table_hbm.at[idx_v.at[pl.ds(j * ch, ch)]],
                ring_v.at[b], gsem[b])
            if j >= look:
                gathers[j - look].wait()
                fire_store(j - look)
        for j in range(n_ch - look, n_ch):
            gathers[j].wait()
            fire_store(j)
        for j in range(max(0, n_ch - nbuf), n_ch):
            stores[j].wait()

    return gk(table, idx)


def _gather_neighbors(table, idx2d):
    N, K = idx2d.shape
    H = table.shape[1]
    flat = idx2d.reshape(-1)
    E = flat.shape[0]
    Ep = ((E + 4095) // 4096) * 4096
    if Ep != E:
        flat = jnp.concatenate([flat, jnp.zeros((Ep - E,), jnp.int32)])
    rows = _gather_rows(table, flat)
    return rows[:E].reshape(N, K, H)


def _edge_body(NB, K, H, Ho, off, has_res, *refs):
    if has_res:
        bj_ref, cn_ref, w2_ref, b2_ref, res_ref, out_ref = refs
    else:
        bj_ref, cn_ref, w2_ref, b2_ref, out_ref = refs
    bj = bj_ref[...][:, :, off:off + H]
    e = jnp.maximum(cn_ref[...][:, None, :] + bj, 0.0)
    h = lax.dot_general(e.reshape(NB * K, H), w2_ref[...],
                        (((1,), (0,)), ((), ())), preferred_element_type=F32)
    r = jnp.max(h.reshape(NB, K, Ho), axis=1) + b2_ref[0:1, :]
    if has_res:
        r = r + res_ref[...]
    out_ref[...] = r


def _edge(bj, cn, W2, b2, off=0, resid=None):
    N, K, Hb = bj.shape
    H = cn.shape[1]
    Ho = W2.shape[1]
    NB = 504 if N % 504 == 0 else 256
    grid = N // NB
    b2b = jnp.broadcast_to(b2[None, :], (8, Ho))
    in_specs = [
        pl.BlockSpec((NB, K, Hb), lambda i: (i, 0, 0)),
        pl.BlockSpec((NB, H), lambda i: (i, 0)),
        pl.BlockSpec((H, Ho), lambda i: (0, 0)),
        pl.BlockSpec((8, Ho), lambda i: (0, 0)),
    ]
    args = [bj, cn, W2, b2b]
    if resid is not None:
        in_specs.append(pl.BlockSpec((NB, Ho), lambda i: (i, 0)))
        args.append(resid)
    return pl.pallas_call(
        functools.partial(_edge_body, NB, K, H, Ho, off, resid is not None),
        grid=(grid,),
        in_specs=in_specs,
        out_specs=pl.BlockSpec((NB, Ho), lambda i: (i, 0)),
        out_shape=jax.ShapeDtypeStruct((N, Ho), F32),
    )(*args)


def _mm_body(relu, x_ref, w_ref, b_ref, out_ref):
    h = lax.dot_general(x_ref[...], w_ref[...], (((1,), (0,)), ((), ())),
                        preferred_element_type=F32) + b_ref[0:1, :]
    if relu:
        h = jnp.maximum(h, 0.0)
    out_ref[...] = h


def _mm(X, W, b, relu=False):
    N, F = X.shape
    Ho = W.shape[1]
    NB = 1008 if N % 1008 == 0 else (512 if N % 512 == 0 else 256)
    grid = N // NB
    b2b = jnp.broadcast_to(b[None, :], (8, Ho))
    return pl.pallas_call(
        functools.partial(_mm_body, relu),
        grid=(grid,),
        in_specs=[
            pl.BlockSpec((NB, F), lambda i: (i, 0)),
            pl.BlockSpec((F, Ho), lambda i: (0, 0)),
            pl.BlockSpec((8, Ho), lambda i: (0, 0)),
        ],
        out_specs=pl.BlockSpec((NB, Ho), lambda i: (i, 0)),
        out_shape=jax.ShapeDtypeStruct((N, Ho), F32),
    )(X, W, b2b)


def _mlp2_body(has_res, x_ref, w1_ref, b1_ref, w2_ref, b2_ref, *rest):
    if has_res:
        res_ref, out_ref = rest
    else:
        (out_ref,) = rest
    h = lax.dot_general(x_ref[...], w1_ref[...], (((1,), (0,)), ((), ())),
                        preferred_element_type=F32) + b1_ref[0:1, :]
    h = jnp.maximum(h, 0.0)
    o = lax.dot_general(h, w2_ref[...], (((1,), (0,)), ((), ())),
                        preferred_element_type=F32) + b2_ref[0:1, :]
    if has_res:
        o = o + res_ref[...]
    out_ref[...] = o


def _mlp2(X, W1, b1, W2, b2, resid=None):
    N, F = X.shape
    Hh = W1.shape[1]
    Ho = W2.shape[1]
    NB = 1008 if N % 1008 == 0 else (512 if N % 512 == 0 else 256)
    grid = N // NB
    b1b = jnp.broadcast_to(b1[None, :], (8, Hh))
    b2b = jnp.broadcast_to(b2[None, :], (8, Ho))
    in_specs = [
        pl.BlockSpec((NB, F), lambda i: (i, 0)),
        pl.BlockSpec((F, Hh), lambda i: (0, 0)),
        pl.BlockSpec((8, Hh), lambda i: (0, 0)),
        pl.BlockSpec((Hh, Ho), lambda i: (0, 0)),
        pl.BlockSpec((8, Ho), lambda i: (0, 0)),
    ]
    args = [X, W1, b1b, W2, b2b]
    if resid is not None:
        in_specs.append(pl.BlockSpec((NB, Ho), lambda i: (i, 0)))
        args.append(resid)
    return pl.pallas_call(
        functools.partial(_mlp2_body, resid is not None),
        grid=(grid,),
        in_specs=in_specs,
        out_specs=pl.BlockSpec((NB, Ho), lambda i: (i, 0)),
        out_shape=jax.ShapeDtypeStruct((N, Ho), F32),
    )(*args)


def _edgeconv(x, idx, W1, b1, W2, b2, resid=None):
    F = x.shape[1]
    H = W1.shape[1]
    W1a, W1b = W1[:F], W1[F:]
    Wcat = jnp.concatenate([W1a - W1b, W1b], axis=1)
    bcat = jnp.concatenate([b1, jnp.zeros((H,), F32)])
    cb = _mm(x, Wcat, bcat)
    cn = cb[:, :H]
    if H % 128 == 0:
        bj = _gather_neighbors(cb[:, H:], idx)
        off = 0
    else:
        bj = _gather_neighbors(cb, idx)
        off = H
    return _edge(bj, cn, W2, b2, off=off, resid=resid)


def kernel(ctx_xyz, ctx_tokens, pred_tokens, params, mask_id):
    p = params

    cxyz = (jnp.repeat(ctx_xyz, CTX_UP, axis=1) +
            jnp.asarray(_N_CTX)).reshape(-1, 3)
    ctok = jnp.repeat(ctx_tokens, CTX_UP, axis=1).reshape(-1, C)

    idx_c = _knn(cxyz, 16)
    x0 = jnp.concatenate([ctok, cxyz], axis=-1)
    x1 = _edgeconv(x0, idx_c, p['ctx1_W1'], p['ctx1_b1'],
                   p['ctx1_W2'], p['ctx1_b2'])
    x1c = jnp.concatenate([x1, cxyz], axis=-1)
    ctx_feat = _edgeconv(x1c, idx_c, p['ctx2_W1'], p['ctx2_b1'],
                         p['ctx2_W2'], p['ctx2_b2'])
    ctx_out = _mlp2(jnp.concatenate([cxyz, ctx_feat], axis=-1),
                    p['def_W1'], p['def_b1'],
                    0.05 * p['def_W2'], 0.05 * p['def_b2'], resid=cxyz)

    pred_tok = pred_tokens[:, mask_id]
    tgt_xyz = _mlp2(pred_tok.reshape(-1, C), p['lat_W1'], p['lat_b1'],
                    p['lat_W2'], p['lat_b2'])[None]
    txyz = (jnp.repeat(jnp.repeat(tgt_xyz, UP, axis=1) + jnp.asarray(_N_X1),
                       UP, axis=1) + jnp.asarray(_N_X2))[:, :TGT_TGT]
    txyz = txyz.reshape(-1, 3)
    ttok = (jnp.repeat(jnp.repeat(pred_tok, UP, axis=1) + jnp.asarray(_N_T1),
                       UP, axis=1) + jnp.asarray(_N_T2))[:, :TGT_TGT]
    ttok = ttok.reshape(-1, C)

    idx16 = _knn(txyz, 16)
    x = _edgeconv(ttok, idx16, p['te1_W1'], p['te1_b1'],
                  p['te1_W2'], p['te1_b2'])
    tgt_feat = _edgeconv(x, idx16, p['te2_W1'], p['te2_b1'],
                         p['te2_W2'], p['te2_b2'])

    g = jnp.linspace(-1.0, 1.0, GRID)
    grid = jnp.stack(jnp.meshgrid(g, g, indexing='ij'), axis=-1).reshape(-1, 2)
    N = tgt_feat.shape[0]
    grid = jnp.tile(grid, (N // grid.shape[0] + 1, 1))[:N].astype(F32)
    xyz = _mlp2(jnp.concatenate([grid, tgt_feat], axis=-1),
                p['f1_W1'], p['f1_b1'], p['f1_W2'], p['f1_b2'])
    xyz = _mlp2(jnp.concatenate([xyz, tgt_feat], axis=-1),
                p['f2_W1'], p['f2_b1'], p['f2_W2'], p['f2_b2'], resid=xyz)

    idx8 = _knn(xyz, 8)
    xf = jnp.concatenate([tgt_feat, xyz], axis=-1)
    xyz = _edgeconv(xf, idx8, p['ref_W1'], p['ref_b1'],
                    p['ref_W2'], p['ref_b2'], resid=xyz)
    return jnp.concatenate([ctx_out, xyz], axis=0)
